# Initial kernel scaffold; baseline (speedup 1.0000x reference)
import jax
import jax.numpy as jnp
from jax.experimental import pallas as pl


def kernel(pred, actual):
    raise NotImplementedError("write your pallas kernel here")


# speedup vs baseline: 1.4687x; 1.4687x over previous
import functools

import jax
import jax.numpy as jnp
from jax import lax
from jax.experimental import pallas as pl
from jax.experimental.pallas import tpu as pltpu

_EPS = 1e-06
_N = 4194304
_ROWS = 4096
_COLS = 1024
_BLK_ROWS = 64
_G = _ROWS // _BLK_ROWS
_CH_ROWS = 8
_C = _CH_ROWS * _COLS
_NCH = _BLK_ROWS // _CH_ROWS
_SHIFTS = (22, 14, 6, 0)


def _round_body(shift, first, lo_ref, fp_ref, pred_ref, act_ref,
                olo_ref, ofp_ref, hist_ref, mse_ref):
    pid = pl.program_id(0)

    @pl.when(pid == 0)
    def _init():
        hist_ref[...] = jnp.zeros_like(hist_ref)
        if first:
            mse_ref[0] = 0.0

    lo0 = lo_ref[0]

    def chunk(i, _):
        p = pred_ref[pl.ds(i * _CH_ROWS, _CH_ROWS), :].reshape(1, _C)
        a = act_ref[pl.ds(i * _CH_ROWS, _CH_ROWS), :].reshape(1, _C)
        p = jnp.clip(p, _EPS, 1.0 - _EPS)
        w = jnp.where(a < 1.0, jnp.float32(20.0), a)
        u = lax.bitcast_convert_type(p, jnp.int32)
        j = (u - lo0) >> shift
        hi = j >> 4
        lob = j & 15
        mhiT = jnp.where(hi == lax.broadcasted_iota(jnp.int32, (16, _C), 0),
                         jnp.float32(1.0), jnp.float32(0.0))
        mlo = jnp.where(lob.reshape(_C, 1)
                        == lax.broadcasted_iota(jnp.int32, (_C, 16), 1),
                        w.reshape(_C, 1), jnp.float32(0.0))
        hist_ref[...] += jnp.dot(mhiT, mlo, preferred_element_type=jnp.float32)
        if first:
            mse_ref[0] += jnp.sum((p - a) ** 2)
        return 0

    lax.fori_loop(0, _NCH, chunk, 0)

    @pl.when(pid == _G - 1)
    def _finish():
        h = hist_ref[...]
        rowsum = jnp.sum(h, axis=1, keepdims=True)
        total = jnp.sum(rowsum)
        if first:
            a_above = jnp.float32(0.0)
            t_target = jnp.float32(0.04) * total
        else:
            a_above = fp_ref[0]
            t_target = fp_ref[1]
        ri = lax.broadcasted_iota(jnp.int32, (16, 16), 0)
        ci = lax.broadcasted_iota(jnp.int32, (16, 16), 1)
        utri = jnp.where(ri > ci, jnp.float32(1.0), jnp.float32(0.0))
        s_in = jnp.dot(h, utri, preferred_element_type=jnp.float32)
        utri2 = jnp.where(ci > ri, jnp.float32(1.0), jnp.float32(0.0))
        row_suf = jnp.dot(utri2, rowsum, preferred_element_type=jnp.float32)
        s_all = a_above + row_suf + s_in
        sel = (s_all <= t_target) & (t_target < s_all + h)
        jidx = ri * 16 + ci
        jstar = jnp.sum(jnp.where(sel, jidx, 0))
        olo_ref[0] = lo0 + (jstar << shift)
        ofp_ref[0] = jnp.sum(jnp.where(sel, s_all, jnp.float32(0.0)))
        ofp_ref[1] = t_target
        ofp_ref[2] = mse_ref[0] if first else fp_ref[2]


def _final_body(lo_ref, fp_ref, pred_ref, act_ref, out_ref, acc_ref):
    pid = pl.program_id(0)

    @pl.when(pid == 0)
    def _init():
        acc_ref[0] = 0.0
        acc_ref[1] = 0.0

    t = lax.bitcast_convert_type(lo_ref[0], jnp.float32)

    def chunk(i, _):
        p = pred_ref[pl.ds(i * _CH_ROWS, _CH_ROWS), :].reshape(1, _C)
        a = act_ref[pl.ds(i * _CH_ROWS, _CH_ROWS), :].reshape(1, _C)
        p = jnp.clip(p, _EPS, 1.0 - _EPS)
        mask = (p > t) & (a == 0.0)
        acc_ref[0] += jnp.sum(jnp.where(mask, -jnp.log(1.0 - p + t),
                                        jnp.float32(0.0)))
        acc_ref[1] += jnp.sum(jnp.where(mask, jnp.float32(1.0),
                                        jnp.float32(0.0)))
        return 0

    lax.fori_loop(0, _NCH, chunk, 0)

    @pl.when(pid == _G - 1)
    def _finish():
        out_ref[0] = fp_ref[2] / jnp.float32(_N) + acc_ref[0] / acc_ref[1]


def _data_spec():
    return pl.BlockSpec((_BLK_ROWS, _COLS), lambda i: (i, 0))


def _smem_spec(n):
    return pl.BlockSpec(memory_space=pltpu.SMEM)


@jax.jit
def kernel(pred, actual):
    p2 = pred.reshape(_ROWS, _COLS)
    a2 = actual.reshape(_ROWS, _COLS)
    lo = jnp.zeros((1,), jnp.int32)
    fp = jnp.zeros((3,), jnp.float32)

    for r, shift in enumerate(_SHIFTS):
        lo, fp = pl.pallas_call(
            functools.partial(_round_body, shift, r == 0),
            grid=(_G,),
            in_specs=[_smem_spec(1), _smem_spec(3), _data_spec(), _data_spec()],
            out_specs=[_smem_spec(1), _smem_spec(3)],
            out_shape=[jax.ShapeDtypeStruct((1,), jnp.int32),
                       jax.ShapeDtypeStruct((3,), jnp.float32)],
            scratch_shapes=[pltpu.VMEM((16, 16), jnp.float32),
                            pltpu.SMEM((1,), jnp.float32)],
        )(lo, fp, p2, a2)

    out = pl.pallas_call(
        _final_body,
        grid=(_G,),
        in_specs=[_smem_spec(1), _smem_spec(3), _data_spec(), _data_spec()],
        out_specs=_smem_spec(1),
        out_shape=jax.ShapeDtypeStruct((1,), jnp.float32),
        scratch_shapes=[pltpu.SMEM((2,), jnp.float32)],
    )(lo, fp, p2, a2)
    return out.reshape(())


# speedup vs baseline: 10.5147x; 7.1591x over previous
import jax
import jax.numpy as jnp
from jax import lax
from jax.experimental import pallas as pl
from jax.experimental.pallas import tpu as pltpu

_EPS = 1e-06
_N = 4194304
_ROWS = 4096
_COLS = 1024
_BLK_ROWS = 128
_G = _ROWS // _BLK_ROWS
_CH_ROWS = 8
_NCH = _BLK_ROWS // _CH_ROWS
_NROUND = 8


def _body(pred_ref, act_ref, out_ref, lo_ref, st_ref, pacc_ref):
    phase = pl.program_id(0)
    i = pl.program_id(1)
    shift = jnp.maximum(26 - 4 * phase, 0)

    @pl.when(jnp.logical_and(phase == 0, i == 0))
    def _init_all():
        lo_ref[0] = 0
        for k in range(6):
            st_ref[k] = 0.0

    @pl.when(i == 0)
    def _init_phase():
        pacc_ref[...] = jnp.zeros_like(pacc_ref)

    lo0 = lo_ref[0]
    t_thr = lax.bitcast_convert_type(lo0, jnp.float32)

    def chunk(c, _):
        p = pred_ref[pl.ds(c * _CH_ROWS, _CH_ROWS), :]
        a = act_ref[pl.ds(c * _CH_ROWS, _CH_ROWS), :]
        p = jnp.clip(p, _EPS, 1.0 - _EPS)

        @pl.when(phase < _NROUND)
        def _hist():
            w = jnp.where(a < 1.0, jnp.float32(20.0), a)
            u = lax.bitcast_convert_type(p, jnp.int32)
            j = (u - lo0) >> shift
            iota = lax.broadcasted_iota(jnp.int32, (_CH_ROWS, 16, _COLS), 1)
            onehot = j[:, None, :] == iota
            pacc_ref[...] += jnp.where(onehot, w[:, None, :], jnp.float32(0.0))

            @pl.when(phase == 0)
            def _mse():
                st_ref[2] += jnp.sum((p - a) ** 2)

        @pl.when(phase == _NROUND)
        def _extra():
            mask = (p > t_thr) & (a == 0.0)
            st_ref[3] += jnp.sum(jnp.where(mask, -jnp.log(1.0 - p + t_thr),
                                           jnp.float32(0.0)))
            st_ref[4] += jnp.sum(jnp.where(mask, jnp.float32(1.0),
                                           jnp.float32(0.0)))

        return 0

    lax.fori_loop(0, _NCH, chunk, 0)

    @pl.when(jnp.logical_and(i == _G - 1, phase < _NROUND))
    def _select():
        h2 = jnp.sum(pacc_ref[...], axis=0)
        h = jnp.sum(h2, axis=1, keepdims=True)
        total = jnp.sum(h)
        a_above = jnp.where(phase == 0, jnp.float32(0.0), st_ref[0])
        t_target = jnp.where(phase == 0, jnp.float32(0.04) * total, st_ref[1])
        ri = lax.broadcasted_iota(jnp.int32, (16, 16), 0)
        ci = lax.broadcasted_iota(jnp.int32, (16, 16), 1)
        utri2 = jnp.where(ci > ri, jnp.float32(1.0), jnp.float32(0.0))
        s_all = a_above + jnp.dot(utri2, h, preferred_element_type=jnp.float32)
        sel = (s_all <= t_target) & (t_target < s_all + h)
        bidx = lax.broadcasted_iota(jnp.int32, (16, 1), 0)
        jstar = jnp.sum(jnp.where(sel, bidx, 0))
        lo_ref[0] = lo0 + (jstar << shift)
        st_ref[0] = jnp.sum(jnp.where(sel, s_all, jnp.float32(0.0)))
        st_ref[1] = t_target

    @pl.when(jnp.logical_and(i == _G - 1, phase == _NROUND))
    def _finish():
        out_ref[0] = st_ref[2] / jnp.float32(_N) + st_ref[3] / st_ref[4]


@jax.jit
def kernel(pred, actual):
    p2 = pred.reshape(_ROWS, _COLS)
    a2 = actual.reshape(_ROWS, _COLS)
    out = pl.pallas_call(
        _body,
        grid=(_NROUND + 1, _G),
        in_specs=[pl.BlockSpec((_BLK_ROWS, _COLS), lambda r, i: (i, 0)),
                  pl.BlockSpec((_BLK_ROWS, _COLS), lambda r, i: (i, 0))],
        out_specs=pl.BlockSpec(memory_space=pltpu.SMEM),
        out_shape=jax.ShapeDtypeStruct((1,), jnp.float32),
        scratch_shapes=[pltpu.SMEM((1,), jnp.int32),
                        pltpu.SMEM((6,), jnp.float32),
                        pltpu.VMEM((_CH_ROWS, 16, _COLS), jnp.float32)],
    )(p2, a2)
    return out.reshape(())


# speedup vs baseline: 13.8319x; 1.3155x over previous
import functools

import jax
import jax.numpy as jnp
from jax import lax
from jax.experimental import pallas as pl
from jax.experimental.pallas import tpu as pltpu
from jax.experimental.pallas import tpu_sc as plsc

_EPS = 1e-06
_N = 4194304
_NC = 2
_NS = 16
_NW = _NC * _NS
_L = 16
_PER_W = _N // _NW
_CHUNK = 8192
_NCHUNK = _PER_W // _CHUNK
_NBIN = 4096
_SHIFTS = (18, 6, 0)


def _sc_round_body(shift, pred_hbm, act_hbm, lo_hbm, out_hbm,
                   hist, pbuf, abuf, red, lobuf):
    wid = lax.axis_index("s") * _NC + lax.axis_index("c")
    base = wid * _PER_W

    pltpu.sync_copy(lo_hbm, lobuf)
    lo_vec = lobuf[...]

    def zero_body(k, _):
        hist[pl.ds(k * _L, _L)] = jnp.zeros((_L,), jnp.float32)
        return 0
    lax.fori_loop(0, _NBIN * _L // _L, zero_body, 0)

    lane = lax.broadcasted_iota(jnp.int32, (_L,), 0)

    for c in range(_NCHUNK):
        off = base + c * _CHUNK
        pltpu.sync_copy(pred_hbm.at[pl.ds(off, _CHUNK)], pbuf)
        pltpu.sync_copy(act_hbm.at[pl.ds(off, _CHUNK)], abuf)

        def body(i, _):
            p = pbuf[pl.ds(i * _L, _L)]
            a = abuf[pl.ds(i * _L, _L)]
            p = jnp.clip(p, _EPS, 1.0 - _EPS)
            u = lax.bitcast_convert_type(p, jnp.int32)
            j = (u - lo_vec) >> shift
            valid = (j >= 0) & (j < _NBIN)
            w = jnp.where(a < 1.0, jnp.float32(20.0), a)
            idx = (j << 4) | lane
            plsc.addupdate_scatter(hist, [idx], w, mask=valid)
            return 0
        lax.fori_loop(0, _CHUNK // _L, body, 0)

    def red_body(g, _):
        bins = lane + g * _L
        bidx = bins << 4
        acc = jnp.zeros((_L,), jnp.float32)
        for l in range(_L):
            acc = acc + plsc.load_gather(hist, [bidx + l])
        red[pl.ds(g * _L, _L)] = acc
        return 0
    lax.fori_loop(0, _NBIN // _L, red_body, 0)

    pltpu.sync_copy(red, out_hbm.at[wid])


def _make_sc_round(shift):
    return pl.kernel(
        functools.partial(_sc_round_body, shift),
        out_type=jax.ShapeDtypeStruct((_NW, _NBIN), jnp.float32),
        mesh=plsc.VectorSubcoreMesh(core_axis_name="c", subcore_axis_name="s"),
        compiler_params=pltpu.CompilerParams(needs_layout_passes=False),
        scratch_types=[
            pltpu.VMEM((_NBIN * _L,), jnp.float32),
            pltpu.VMEM((_CHUNK,), jnp.float32),
            pltpu.VMEM((_CHUNK,), jnp.float32),
            pltpu.VMEM((_NBIN,), jnp.float32),
            pltpu.VMEM((_L,), jnp.int32),
        ],
    )


def _make_select(shift, first):
    def body(lo_ref, fp_ref, part_ref, olo_ref, ofp_ref):
        h = jnp.sum(part_ref[...], axis=0)
        rowsum = jnp.sum(h, axis=1, keepdims=True)
        total = jnp.sum(rowsum)
        if first:
            a_above = jnp.float32(0.0)
            t_target = jnp.float32(0.04) * total
        else:
            a_above = fp_ref[0]
            t_target = fp_ref[1]
        ri32 = lax.broadcasted_iota(jnp.int32, (32, 32), 0)
        ci32 = lax.broadcasted_iota(jnp.int32, (32, 32), 1)
        u32 = jnp.where(ci32 > ri32, jnp.float32(1.0), jnp.float32(0.0))
        ri128 = lax.broadcasted_iota(jnp.int32, (128, 128), 0)
        ci128 = lax.broadcasted_iota(jnp.int32, (128, 128), 1)
        u128 = jnp.where(ri128 > ci128, jnp.float32(1.0), jnp.float32(0.0))
        s_in = jnp.dot(h, u128, preferred_element_type=jnp.float32)
        row_suf = jnp.dot(u32, rowsum, preferred_element_type=jnp.float32)
        s_all = a_above + row_suf + s_in
        sel = (s_all <= t_target) & (t_target < s_all + h)
        rr = lax.broadcasted_iota(jnp.int32, (32, 128), 0)
        cc = lax.broadcasted_iota(jnp.int32, (32, 128), 1)
        jstar = jnp.sum(jnp.where(sel, rr * 128 + cc, 0))
        olo_ref[0] = lo_ref[0] + (jstar << shift)
        ofp_ref[0] = jnp.sum(jnp.where(sel, s_all, jnp.float32(0.0)))
        ofp_ref[1] = t_target

    return pl.pallas_call(
        body,
        in_specs=[pl.BlockSpec(memory_space=pltpu.SMEM),
                  pl.BlockSpec(memory_space=pltpu.SMEM),
                  pl.BlockSpec((_NW, 32, 128), lambda: (0, 0, 0))],
        out_specs=[pl.BlockSpec(memory_space=pltpu.SMEM),
                   pl.BlockSpec(memory_space=pltpu.SMEM)],
        out_shape=[jax.ShapeDtypeStruct((1,), jnp.int32),
                   jax.ShapeDtypeStruct((2,), jnp.float32)],
    )


_ROWS = 4096
_COLS = 1024
_BLK_ROWS = 128
_G = _ROWS // _BLK_ROWS
_CH_ROWS = 8
_NCH = _BLK_ROWS // _CH_ROWS


def _final_body(lo_ref, pred_ref, act_ref, out_ref, acc_ref):
    i = pl.program_id(0)

    @pl.when(i == 0)
    def _init():
        for k in range(3):
            acc_ref[k] = 0.0

    t = lax.bitcast_convert_type(lo_ref[0], jnp.float32)

    def chunk(c, _):
        p = pred_ref[pl.ds(c * _CH_ROWS, _CH_ROWS), :]
        a = act_ref[pl.ds(c * _CH_ROWS, _CH_ROWS), :]
        p = jnp.clip(p, _EPS, 1.0 - _EPS)
        mask = (p > t) & (a == 0.0)
        acc_ref[0] += jnp.sum(jnp.where(mask, -jnp.log(1.0 - p + t),
                                        jnp.float32(0.0)))
        acc_ref[1] += jnp.sum(jnp.where(mask, jnp.float32(1.0),
                                        jnp.float32(0.0)))
        acc_ref[2] += jnp.sum((p - a) ** 2)
        return 0

    lax.fori_loop(0, _NCH, chunk, 0)

    @pl.when(i == _G - 1)
    def _finish():
        out_ref[0] = acc_ref[2] / jnp.float32(_N) + acc_ref[0] / acc_ref[1]


_final_call = pl.pallas_call(
    _final_body,
    grid=(_G,),
    in_specs=[pl.BlockSpec(memory_space=pltpu.SMEM),
              pl.BlockSpec((_BLK_ROWS, _COLS), lambda i: (i, 0)),
              pl.BlockSpec((_BLK_ROWS, _COLS), lambda i: (i, 0))],
    out_specs=pl.BlockSpec(memory_space=pltpu.SMEM),
    out_shape=jax.ShapeDtypeStruct((1,), jnp.float32),
    scratch_shapes=[pltpu.SMEM((3,), jnp.float32)],
)


@jax.jit
def kernel(pred, actual):
    lo = jnp.zeros((1,), jnp.int32)
    fp = jnp.zeros((2,), jnp.float32)
    for r, shift in enumerate(_SHIFTS):
        lo_vec = jnp.broadcast_to(lo, (_L,)).astype(jnp.int32)
        part = _make_sc_round(shift)(pred, actual, lo_vec)
        lo, fp = _make_select(shift, r == 0)(
            lo, fp, part.reshape(_NW, 32, 128))
    out = _final_call(lo, pred.reshape(_ROWS, _COLS),
                      actual.reshape(_ROWS, _COLS))
    return out.reshape(())


# speedup vs baseline: 28.1666x; 2.0364x over previous
import functools

import jax
import jax.numpy as jnp
from jax import lax
from jax.experimental import pallas as pl
from jax.experimental.pallas import tpu as pltpu
from jax.experimental.pallas import tpu_sc as plsc

_EPS = 1e-06
_N = 4194304
_NC = 2
_NS = 16
_NW = _NC * _NS
_L = 16
_PER_W = _N // _NW
_CHUNK = 8192
_NCHUNK = _PER_W // _CHUNK
_NBIN = 4096
_SHIFTS = (18, 6, 0)


def _sc_round_body(shift, first, pred_hbm, act_hbm, lo_hbm, out_hbm,
                   hist, pbuf, abuf, red, lobuf, semp, sema):
    wid = lax.axis_index("s") * _NC + lax.axis_index("c")
    base = wid * _PER_W

    pltpu.sync_copy(lo_hbm, lobuf)
    lo_vec = lobuf[...]

    @plsc.parallel_loop(0, _NBIN * _L // _L, unroll=4)
    def _zero(k):
        hist[pl.ds(k * _L, _L)] = jnp.zeros((_L,), jnp.float32)

    lane = lax.broadcasted_iota(jnp.int32, (_L,), 0)

    def start(c):
        b = c % 2
        off = base + c * _CHUNK
        hp = pltpu.async_copy(pred_hbm.at[pl.ds(off, _CHUNK)],
                              pbuf.at[b], semp.at[b])
        ha = pltpu.async_copy(act_hbm.at[pl.ds(off, _CHUNK)],
                              abuf.at[b], sema.at[b])
        return hp, ha

    pend = start(0)
    for c in range(_NCHUNK):
        b = c % 2
        pend[0].wait()
        pend[1].wait()
        if c + 1 < _NCHUNK:
            pend = start(c + 1)

        @plsc.parallel_loop(0, _CHUNK // _L, unroll=4)
        def _scatter(i):
            p = pbuf[b, pl.ds(i * _L, _L)]
            a = abuf[b, pl.ds(i * _L, _L)]
            p = jnp.clip(p, _EPS, 1.0 - _EPS)
            u = lax.bitcast_convert_type(p, jnp.int32)
            w = jnp.where(a < 1.0, jnp.float32(20.0), a)
            if first:
                idx = ((u >> shift) << 4) | lane
                plsc.addupdate_scatter(hist, [idx], w)
            else:
                j = (u - lo_vec) >> shift
                valid = (j >= 0) & (j < _NBIN)
                idx = (j << 4) | lane
                plsc.addupdate_scatter(hist, [idx], w, mask=valid)

    @plsc.parallel_loop(0, _NBIN // _L, unroll=2)
    def _reduce(g):
        bidx = (lane + g * _L) << 4
        acc = jnp.zeros((_L,), jnp.float32)
        for l in range(_L):
            acc = acc + plsc.load_gather(hist, [bidx + l])
        red[pl.ds(g * _L, _L)] = acc

    pltpu.sync_copy(red, out_hbm.at[wid])


def _make_sc_round(shift, first):
    return pl.kernel(
        functools.partial(_sc_round_body, shift, first),
        out_type=jax.ShapeDtypeStruct((_NW, _NBIN), jnp.float32),
        mesh=plsc.VectorSubcoreMesh(core_axis_name="c", subcore_axis_name="s"),
        compiler_params=pltpu.CompilerParams(needs_layout_passes=False),
        scratch_types=[
            pltpu.VMEM((_NBIN * _L,), jnp.float32),
            pltpu.VMEM((2, _CHUNK), jnp.float32),
            pltpu.VMEM((2, _CHUNK), jnp.float32),
            pltpu.VMEM((_NBIN,), jnp.float32),
            pltpu.VMEM((_L,), jnp.int32),
            pltpu.SemaphoreType.DMA((2,)),
            pltpu.SemaphoreType.DMA((2,)),
        ],
    )


def _make_select(shift, first):
    def body(lo_ref, fp_ref, part_ref, olo_ref, ofp_ref):
        h = jnp.sum(part_ref[...], axis=0)
        rowsum = jnp.sum(h, axis=1, keepdims=True)
        total = jnp.sum(rowsum)
        if first:
            a_above = jnp.float32(0.0)
            t_target = jnp.float32(0.04) * total
        else:
            a_above = fp_ref[0]
            t_target = fp_ref[1]
        ri32 = lax.broadcasted_iota(jnp.int32, (32, 32), 0)
        ci32 = lax.broadcasted_iota(jnp.int32, (32, 32), 1)
        u32 = jnp.where(ci32 > ri32, jnp.float32(1.0), jnp.float32(0.0))
        ri128 = lax.broadcasted_iota(jnp.int32, (128, 128), 0)
        ci128 = lax.broadcasted_iota(jnp.int32, (128, 128), 1)
        u128 = jnp.where(ri128 > ci128, jnp.float32(1.0), jnp.float32(0.0))
        s_in = jnp.dot(h, u128, preferred_element_type=jnp.float32)
        row_suf = jnp.dot(u32, rowsum, preferred_element_type=jnp.float32)
        s_all = a_above + row_suf + s_in
        sel = (s_all <= t_target) & (t_target < s_all + h)
        rr = lax.broadcasted_iota(jnp.int32, (32, 128), 0)
        cc = lax.broadcasted_iota(jnp.int32, (32, 128), 1)
        jstar = jnp.sum(jnp.where(sel, rr * 128 + cc, 0))
        olo_ref[0] = lo_ref[0] + (jstar << shift)
        ofp_ref[0] = jnp.sum(jnp.where(sel, s_all, jnp.float32(0.0)))
        ofp_ref[1] = t_target

    return pl.pallas_call(
        body,
        in_specs=[pl.BlockSpec(memory_space=pltpu.SMEM),
                  pl.BlockSpec(memory_space=pltpu.SMEM),
                  pl.BlockSpec((_NW, 32, 128), lambda: (0, 0, 0))],
        out_specs=[pl.BlockSpec(memory_space=pltpu.SMEM),
                   pl.BlockSpec(memory_space=pltpu.SMEM)],
        out_shape=[jax.ShapeDtypeStruct((1,), jnp.int32),
                   jax.ShapeDtypeStruct((2,), jnp.float32)],
    )


_ROWS = 4096
_COLS = 1024
_BLK_ROWS = 128
_G = _ROWS // _BLK_ROWS
_CH_ROWS = 8
_NCH = _BLK_ROWS // _CH_ROWS


def _final_body(lo_ref, pred_ref, act_ref, out_ref, acc_ref):
    i = pl.program_id(0)

    @pl.when(i == 0)
    def _init():
        for k in range(3):
            acc_ref[k] = 0.0

    t = lax.bitcast_convert_type(lo_ref[0], jnp.float32)

    def chunk(c, _):
        p = pred_ref[pl.ds(c * _CH_ROWS, _CH_ROWS), :]
        a = act_ref[pl.ds(c * _CH_ROWS, _CH_ROWS), :]
        p = jnp.clip(p, _EPS, 1.0 - _EPS)
        mask = (p > t) & (a == 0.0)
        acc_ref[0] += jnp.sum(jnp.where(mask, -jnp.log(1.0 - p + t),
                                        jnp.float32(0.0)))
        acc_ref[1] += jnp.sum(jnp.where(mask, jnp.float32(1.0),
                                        jnp.float32(0.0)))
        acc_ref[2] += jnp.sum((p - a) ** 2)
        return 0

    lax.fori_loop(0, _NCH, chunk, 0)

    @pl.when(i == _G - 1)
    def _finish():
        out_ref[0] = acc_ref[2] / jnp.float32(_N) + acc_ref[0] / acc_ref[1]


_final_call = pl.pallas_call(
    _final_body,
    grid=(_G,),
    in_specs=[pl.BlockSpec(memory_space=pltpu.SMEM),
              pl.BlockSpec((_BLK_ROWS, _COLS), lambda i: (i, 0)),
              pl.BlockSpec((_BLK_ROWS, _COLS), lambda i: (i, 0))],
    out_specs=pl.BlockSpec(memory_space=pltpu.SMEM),
    out_shape=jax.ShapeDtypeStruct((1,), jnp.float32),
    scratch_shapes=[pltpu.SMEM((3,), jnp.float32)],
)


@jax.jit
def kernel(pred, actual):
    lo = jnp.zeros((1,), jnp.int32)
    fp = jnp.zeros((2,), jnp.float32)
    for r, shift in enumerate(_SHIFTS):
        lo_vec = jnp.broadcast_to(lo, (_L,)).astype(jnp.int32)
        part = _make_sc_round(shift, r == 0)(pred, actual, lo_vec)
        lo, fp = _make_select(shift, r == 0)(
            lo, fp, part.reshape(_NW, 32, 128))
    out = _final_call(lo, pred.reshape(_ROWS, _COLS),
                      actual.reshape(_ROWS, _COLS))
    return out.reshape(())


# speedup vs baseline: 28.9246x; 1.0269x over previous
import functools

import jax
import jax.numpy as jnp
from jax import lax
from jax.experimental import pallas as pl
from jax.experimental.pallas import tpu as pltpu
from jax.experimental.pallas import tpu_sc as plsc

_EPS = 1e-06
_N = 4194304
_NC = 2
_NS = 16
_NW = _NC * _NS
_L = 16
_PER_W = _N // _NW
_CHUNK = 8192
_NCHUNK = _PER_W // _CHUNK
_NBIN = 4096
_SHIFTS = (18, 6, 0)


def _sc_round_body(shift, first, pred_hbm, act_hbm, lo_hbm, out_hbm,
                   hist, pbuf, abuf, red, lobuf, semp, sema):
    wid = lax.axis_index("s") * _NC + lax.axis_index("c")
    base = wid * _PER_W

    pltpu.sync_copy(lo_hbm, lobuf)
    lo_vec = lobuf[...]

    @plsc.parallel_loop(0, _NBIN * _L // _L, unroll=8)
    def _zero(k):
        hist[pl.ds(k * _L, _L)] = jnp.zeros((_L,), jnp.float32)

    lane = lax.broadcasted_iota(jnp.int32, (_L,), 0)

    def start(c):
        b = c % 2
        off = base + c * _CHUNK
        hp = pltpu.async_copy(pred_hbm.at[pl.ds(off, _CHUNK)],
                              pbuf.at[b], semp.at[b])
        ha = pltpu.async_copy(act_hbm.at[pl.ds(off, _CHUNK)],
                              abuf.at[b], sema.at[b])
        return hp, ha

    pend = start(0)
    for c in range(_NCHUNK):
        b = c % 2
        pend[0].wait()
        pend[1].wait()
        if c + 1 < _NCHUNK:
            pend = start(c + 1)

        @plsc.parallel_loop(0, _CHUNK // _L, unroll=8)
        def _scatter(i):
            p = pbuf[b, pl.ds(i * _L, _L)]
            a = abuf[b, pl.ds(i * _L, _L)]
            p = jnp.clip(p, _EPS, 1.0 - _EPS)
            u = lax.bitcast_convert_type(p, jnp.int32)
            w = jnp.where(a < 1.0, jnp.float32(20.0), a)
            if first:
                idx = ((u >> shift) << 4) | lane
                plsc.addupdate_scatter(hist, [idx], w)
            else:
                j = (u - lo_vec) >> shift
                valid = (j >= 0) & (j < _NBIN)
                idx = (j << 4) | lane
                plsc.addupdate_scatter(hist, [idx], w, mask=valid)

    @plsc.parallel_loop(0, _NBIN // _L, unroll=2)
    def _reduce(g):
        bidx = (lane + g * _L) << 4
        acc = jnp.zeros((_L,), jnp.float32)
        for l in range(_L):
            acc = acc + plsc.load_gather(hist, [bidx + l])
        red[pl.ds(g * _L, _L)] = acc

    pltpu.sync_copy(red, out_hbm.at[wid])


def _make_sc_round(shift, first):
    return pl.kernel(
        functools.partial(_sc_round_body, shift, first),
        out_type=jax.ShapeDtypeStruct((_NW, _NBIN), jnp.float32),
        mesh=plsc.VectorSubcoreMesh(core_axis_name="c", subcore_axis_name="s"),
        compiler_params=pltpu.CompilerParams(needs_layout_passes=False),
        scratch_types=[
            pltpu.VMEM((_NBIN * _L,), jnp.float32),
            pltpu.VMEM((2, _CHUNK), jnp.float32),
            pltpu.VMEM((2, _CHUNK), jnp.float32),
            pltpu.VMEM((_NBIN,), jnp.float32),
            pltpu.VMEM((_L,), jnp.int32),
            pltpu.SemaphoreType.DMA((2,)),
            pltpu.SemaphoreType.DMA((2,)),
        ],
    )


_SR = _NBIN // 128


def _make_select(shift, first):
    def body(lo_ref, fp_ref, part_ref, olo_ref, ofp_ref):
        h = jnp.sum(part_ref[...], axis=0)
        rowsum = jnp.sum(h, axis=1, keepdims=True)
        total = jnp.sum(rowsum)
        if first:
            a_above = jnp.float32(0.0)
            t_target = jnp.float32(0.04) * total
        else:
            a_above = fp_ref[0]
            t_target = fp_ref[1]
        rs = lax.broadcasted_iota(jnp.int32, (_SR, _SR), 0)
        cs = lax.broadcasted_iota(jnp.int32, (_SR, _SR), 1)
        usr = jnp.where(cs > rs, jnp.float32(1.0), jnp.float32(0.0))
        r128 = lax.broadcasted_iota(jnp.int32, (128, 128), 0)
        c128 = lax.broadcasted_iota(jnp.int32, (128, 128), 1)
        u128 = jnp.where(r128 > c128, jnp.float32(1.0), jnp.float32(0.0))
        s_in = jnp.dot(h, u128, preferred_element_type=jnp.float32)
        row_suf = jnp.dot(usr, rowsum, preferred_element_type=jnp.float32)
        s_all = a_above + row_suf + s_in
        sel = (s_all <= t_target) & (t_target < s_all + h)
        rr = lax.broadcasted_iota(jnp.int32, (_SR, 128), 0)
        cc = lax.broadcasted_iota(jnp.int32, (_SR, 128), 1)
        jstar = jnp.sum(jnp.where(sel, rr * 128 + cc, 0))
        olo_ref[0] = lo_ref[0] + (jstar << shift)
        ofp_ref[0] = jnp.sum(jnp.where(sel, s_all, jnp.float32(0.0)))
        ofp_ref[1] = t_target

    return pl.pallas_call(
        body,
        in_specs=[pl.BlockSpec(memory_space=pltpu.SMEM),
                  pl.BlockSpec(memory_space=pltpu.SMEM),
                  pl.BlockSpec((_NW, _SR, 128), lambda: (0, 0, 0))],
        out_specs=[pl.BlockSpec(memory_space=pltpu.SMEM),
                   pl.BlockSpec(memory_space=pltpu.SMEM)],
        out_shape=[jax.ShapeDtypeStruct((1,), jnp.int32),
                   jax.ShapeDtypeStruct((2,), jnp.float32)],
    )


_ROWS = 4096
_COLS = 1024
_BLK_ROWS = 128
_G = _ROWS // _BLK_ROWS
_CH_ROWS = 8
_NCH = _BLK_ROWS // _CH_ROWS


def _final_body(lo_ref, pred_ref, act_ref, out_ref, acc_ref):
    i = pl.program_id(0)

    @pl.when(i == 0)
    def _init():
        for k in range(3):
            acc_ref[k] = 0.0

    t = lax.bitcast_convert_type(lo_ref[0], jnp.float32)

    def chunk(c, _):
        p = pred_ref[pl.ds(c * _CH_ROWS, _CH_ROWS), :]
        a = act_ref[pl.ds(c * _CH_ROWS, _CH_ROWS), :]
        p = jnp.clip(p, _EPS, 1.0 - _EPS)
        mask = (p > t) & (a == 0.0)
        acc_ref[0] += jnp.sum(jnp.where(mask, -jnp.log(1.0 - p + t),
                                        jnp.float32(0.0)))
        acc_ref[1] += jnp.sum(jnp.where(mask, jnp.float32(1.0),
                                        jnp.float32(0.0)))
        acc_ref[2] += jnp.sum((p - a) ** 2)
        return 0

    lax.fori_loop(0, _NCH, chunk, 0)

    @pl.when(i == _G - 1)
    def _finish():
        out_ref[0] = acc_ref[2] / jnp.float32(_N) + acc_ref[0] / acc_ref[1]


_final_call = pl.pallas_call(
    _final_body,
    grid=(_G,),
    in_specs=[pl.BlockSpec(memory_space=pltpu.SMEM),
              pl.BlockSpec((_BLK_ROWS, _COLS), lambda i: (i, 0)),
              pl.BlockSpec((_BLK_ROWS, _COLS), lambda i: (i, 0))],
    out_specs=pl.BlockSpec(memory_space=pltpu.SMEM),
    out_shape=jax.ShapeDtypeStruct((1,), jnp.float32),
    scratch_shapes=[pltpu.SMEM((3,), jnp.float32)],
)


@jax.jit
def kernel(pred, actual):
    lo = jnp.zeros((1,), jnp.int32)
    fp = jnp.zeros((2,), jnp.float32)
    for r, shift in enumerate(_SHIFTS):
        lo_vec = jnp.broadcast_to(lo, (_L,)).astype(jnp.int32)
        part = _make_sc_round(shift, r == 0)(pred, actual, lo_vec)
        lo, fp = _make_select(shift, r == 0)(
            lo, fp, part.reshape(_NW, _SR, 128))
    out = _final_call(lo, pred.reshape(_ROWS, _COLS),
                      actual.reshape(_ROWS, _COLS))
    return out.reshape(())
